# baseline (speedup 1.0000x reference)
import functools

import jax
import jax.numpy as jnp
from jax import lax
from jax.experimental import pallas as pl
from jax.experimental.pallas import tpu as pltpu
from jax.experimental.pallas import tpu_sc as plsc

N = 10000
E = 320000
D = 128
DP = 144
NC = 2
NS = 16
L = 16
NW = NC * NS
EPW = E // NW
K = 80
RPW = N // NS

_MAXNORM = 1.0 - 4e-3

_mesh = plsc.VectorSubcoreMesh(core_axis_name="c", subcore_axis_name="s")


def _artanh(v):
    c = jnp.clip(v, -1.0 + 1e-7, 1.0 - 1e-7)
    return 0.5 * jnp.log((1.0 + c) / (1.0 - c))


def _tanh15(v):
    return jnp.tanh(jnp.clip(v, -15.0, 15.0))


def _rownorm(v):
    return jnp.clip(
        jnp.sqrt(jnp.sum(v * v, axis=-1, keepdims=True)), 1e-15, None)


NP = 10240
BN = 1024


def _node_stage_body(x_ref, w_ref, a_ref, hpad_ref, q_ref, pa_ref, pb_ref):
    x = x_ref[...]
    w = w_ref[...]
    av = a_ref[...]
    xn = _rownorm(x)
    mx = jnp.dot(x, w, preferred_element_type=jnp.float32)
    mxn = _rownorm(mx)
    h1 = _tanh15(mxn / xn * _artanh(xn)) * mx / mxn
    n1 = _rownorm(h1)
    h2 = jnp.where(n1 > _MAXNORM, h1 / n1 * _MAXNORM, h1)
    pn = _rownorm(h2)
    h = h2 / pn * _artanh(pn)
    q_ref[...] = jnp.sum(h * h, axis=-1)
    pa_ref[...] = jnp.sum(h * av[:, :D], axis=-1)
    pb_ref[...] = jnp.sum(h * av[:, D:], axis=-1)
    ones = jnp.ones((h.shape[0], 1), jnp.float32)
    zeros = jnp.zeros((h.shape[0], DP - D - 1), jnp.float32)
    hpad_ref[...] = jnp.concatenate([h, ones, zeros], axis=1)


_node_stage = pl.pallas_call(
    _node_stage_body,
    grid=(NP // BN,),
    in_specs=[
        pl.BlockSpec((BN, D), lambda i: (i, 0)),
        pl.BlockSpec((D, D), lambda i: (0, 0)),
        pl.BlockSpec((1, 2 * D), lambda i: (0, 0)),
    ],
    out_specs=[
        pl.BlockSpec((BN, DP), lambda i: (i, 0)),
        pl.BlockSpec((BN,), lambda i: (i,)),
        pl.BlockSpec((BN,), lambda i: (i,)),
        pl.BlockSpec((BN,), lambda i: (i,)),
    ],
    out_shape=[
        jax.ShapeDtypeStruct((NP, DP), jnp.float32),
        jax.ShapeDtypeStruct((NP,), jnp.float32),
        jax.ShapeDtypeStruct((NP,), jnp.float32),
        jax.ShapeDtypeStruct((NP,), jnp.float32),
    ],
)


@functools.partial(
    pl.kernel,
    out_type=[
        jax.ShapeDtypeStruct((E,), jnp.float32),
        jax.ShapeDtypeStruct((E,), jnp.float32),
    ],
    mesh=_mesh,
    scratch_types=[
        pltpu.VMEM((NP,), jnp.float32),
        pltpu.VMEM((NP,), jnp.float32),
        pltpu.VMEM((NP,), jnp.float32),
        pltpu.VMEM((EPW,), jnp.int32),
        pltpu.VMEM((EPW,), jnp.int32),
        pltpu.VMEM((EPW,), jnp.float32),
        pltpu.VMEM((EPW,), jnp.float32),
        pltpu.SemaphoreType.DMA,
    ],
    compiler_params=pltpu.CompilerParams(needs_layout_passes=False),
)
def _edge_gather(q_hbm, pa_hbm, pb_hbm, e0_hbm, e1_hbm, r2_hbm, t_hbm,
                 qv, pav, pbv, i0v, i1v, r2v, tv, sem):
    wid = lax.axis_index("s") * NC + lax.axis_index("c")
    base = wid * EPW
    pltpu.async_copy(q_hbm, qv, sem)
    pltpu.async_copy(pa_hbm, pav, sem)
    pltpu.async_copy(pb_hbm, pbv, sem)
    pltpu.async_copy(e0_hbm.at[pl.ds(base, EPW)], i0v, sem)
    pltpu.async_copy(e1_hbm.at[pl.ds(base, EPW)], i1v, sem)
    pltpu.make_async_copy(q_hbm, qv, sem).wait()
    pltpu.make_async_copy(pa_hbm, pav, sem).wait()
    pltpu.make_async_copy(pb_hbm, pbv, sem).wait()
    pltpu.make_async_copy(e0_hbm.at[pl.ds(base, EPW)], i0v, sem).wait()
    pltpu.make_async_copy(e1_hbm.at[pl.ds(base, EPW)], i1v, sem).wait()

    def body(i5, carry):
        for u in range(5):
            i = i5 * 5 + u
            sl = pl.ds(i * L, L)
            i0 = i0v[sl]
            i1 = i1v[sl]
            q0 = plsc.load_gather(qv, [i0])
            q1 = plsc.load_gather(qv, [i1])
            p0 = plsc.load_gather(pav, [i0])
            p1 = plsc.load_gather(pbv, [i1])
            r2v[sl] = q0 + q1
            tv[sl] = p0 + p1
        return carry

    lax.fori_loop(0, EPW // L // 5, body, 0)
    pltpu.sync_copy(r2v, r2_hbm.at[pl.ds(base, EPW)])
    pltpu.sync_copy(tv, t_hbm.at[pl.ds(base, EPW)])


def _edge_weight_body(r2_ref, t_ref, ee_ref):
    r2 = r2_ref[...]
    t = t_ref[...]
    un = jnp.sqrt(r2)
    r = jnp.maximum(un, 1e-15)
    em_scale = _tanh15(r) / r
    nu = jnp.maximum(em_scale * un, 1e-15)
    p_scale = jnp.where(nu > _MAXNORM, _MAXNORM / nu, 1.0)
    alpha = em_scale * p_scale
    rho = jnp.maximum(alpha * un, 1e-15)
    mx = alpha * t
    mxn = jnp.maximum(jnp.abs(mx), 1e-15)
    s = _tanh15(mxn / rho * _artanh(rho)) * mx / mxn
    bign = jnp.sqrt(jnp.sum(s * s))
    proj_scale = jnp.where(bign > _MAXNORM, _MAXNORM / bign, 1.0)
    pn = jnp.maximum(bign * proj_scale, 1e-15)
    beta = _artanh(pn) / pn * proj_scale
    sf = s * beta
    gelu = 0.5 * sf * (1.0 + lax.erf(sf / jnp.sqrt(2.0).astype(jnp.float32)))
    ee_ref[...] = jnp.exp(-gelu)


_edge_weight = pl.pallas_call(
    _edge_weight_body,
    out_shape=jax.ShapeDtypeStruct((E // D, D), jnp.float32),
)


NCHK = EPW // K


@functools.partial(
    pl.kernel,
    out_type=jax.ShapeDtypeStruct((NC, N, DP), jnp.float32),
    mesh=_mesh,
    scratch_types=(
        [pltpu.VMEM((K,), jnp.int32)] * 3
        + [pltpu.VMEM((K,), jnp.int32)] * 3
        + [pltpu.VMEM((K,), jnp.float32)] * 3
        + [pltpu.VMEM((K,), jnp.int32)] * 3
        + [pltpu.VMEM((K, DP), jnp.float32)] * 3
        + [pltpu.VMEM_SHARED((N, DP), jnp.float32)]
        + [pltpu.SemaphoreType.DMA] * 9
    ),
    compiler_params=pltpu.CompilerParams(
        needs_layout_passes=False, use_tc_tiling_on_sc=False),
)
def _spmm(hpad_hbm, e0_hbm, e1_hbm, ee_hbm, zero_hbm, out_hbm,
          i0_0, i0_1, i0_2, i1_0, i1_1, i1_2, ee_0, ee_1, ee_2,
          si_0, si_1, si_2, rw_0, rw_1, rw_2, acc,
          g_0, g_1, g_2, c_0, c_1, c_2, s_0, s_1, s_2):
    cid = lax.axis_index("c")
    sid = lax.axis_index("s")
    base = (sid * NC + cid) * EPW
    pltpu.sync_copy(zero_hbm.at[pl.ds(sid * RPW, RPW)],
                    acc.at[pl.ds(sid * RPW, RPW)])
    plsc.subcore_barrier()

    sets = [
        dict(i0=i0_0, i1=i1_0, ee=ee_0, si=si_0, rw=rw_0, g=g_0, c=c_0,
             s=s_0),
        dict(i0=i0_1, i1=i1_1, ee=ee_1, si=si_1, rw=rw_1, g=g_1, c=c_1,
             s=s_1),
        dict(i0=i0_2, i1=i1_2, ee=ee_2, si=si_2, rw=rw_2, g=g_2, c=c_2,
             s=s_2),
    ]

    def issue_idx(k, st):
        off = base + k * K
        pltpu.async_copy(e0_hbm.at[pl.ds(off, K)], st["i0"], st["s"])
        pltpu.async_copy(e1_hbm.at[pl.ds(off, K)], st["i1"], st["s"])
        pltpu.async_copy(ee_hbm.at[pl.ds(off, K)], st["ee"], st["s"])

    def wait_idx(st):
        pltpu.make_async_copy(e0_hbm.at[pl.ds(0, K)], st["i0"], st["s"]).wait()
        pltpu.make_async_copy(e1_hbm.at[pl.ds(0, K)], st["i1"], st["s"]).wait()
        pltpu.make_async_copy(ee_hbm.at[pl.ds(0, K)], st["ee"], st["s"]).wait()

    def wait_scatter(st):
        pltpu.make_async_copy(st["rw"], acc.at[st["si"]], st["c"]).wait()

    def slot(k, cur, nxt, prv):
        @pl.when(k + 1 < NCHK)
        def _():
            wait_idx(nxt)

            @pl.when(k >= 2)
            def _():
                wait_scatter(nxt)

            pltpu.async_copy(hpad_hbm.at[nxt["i1"]], nxt["rw"], nxt["g"])

        buf = cur["rw"]
        ee = cur["ee"]
        pltpu.make_async_copy(hpad_hbm.at[cur["i1"]], buf, cur["g"]).wait()

        def _scale(r8, c2):
            for u in range(8):
                r = r8 * 8 + u
                eb = plsc.load_gather(ee, [jnp.full((L,), 0, jnp.int32) + r])
                for j in range(DP // L):
                    sl = pl.ds(j * L, L)
                    buf[r, sl] = buf[r, sl] * eb
            return c2

        lax.fori_loop(0, K // 8, _scale, 0)

        for v in range(K // L):
            sl = pl.ds(v * L, L)
            cur["si"][sl] = cur["i0"][sl]

        @pl.when(k + 2 < NCHK)
        def _():
            issue_idx(k + 2, prv)

        pltpu.async_copy(buf, acc.at[cur["si"]], cur["c"], add=True)

    issue_idx(0, sets[0])
    issue_idx(1, sets[1])
    wait_idx(sets[0])
    pltpu.async_copy(hpad_hbm.at[sets[0]["i1"]], sets[0]["rw"], sets[0]["g"])

    def triple(k3, carry):
        k = 3 * k3
        slot(k, sets[0], sets[1], sets[2])

        @pl.when(k + 1 < NCHK)
        def _():
            slot(k + 1, sets[1], sets[2], sets[0])

        @pl.when(k + 2 < NCHK)
        def _():
            slot(k + 2, sets[2], sets[0], sets[1])

        return carry

    lax.fori_loop(0, (NCHK + 2) // 3, triple, 0)
    wait_scatter(sets[(NCHK - 3) % 3])
    wait_scatter(sets[(NCHK - 2) % 3])
    wait_scatter(sets[(NCHK - 1) % 3])
    plsc.subcore_barrier()
    pltpu.sync_copy(acc.at[pl.ds(sid * RPW, RPW)],
                    out_hbm.at[cid, pl.ds(sid * RPW, RPW)])


def _finalize_body(acc_ref, out_ref):
    A = acc_ref[0] + acc_ref[1]
    hp = A[:, :D]
    col = lax.broadcasted_iota(jnp.int32, A.shape, 1)
    rs = jnp.sum(jnp.where(col == D, A, 0.0), axis=-1, keepdims=True)
    v = jax.nn.relu(hp / rs)
    un = _rownorm(v)
    em = _tanh15(un) * v / un
    n1 = _rownorm(em)
    out_ref[...] = jnp.where(n1 > _MAXNORM, em / n1 * _MAXNORM, em)


_finalize = pl.pallas_call(
    _finalize_body,
    out_shape=jax.ShapeDtypeStruct((N, D), jnp.float32),
)


def kernel(x, edge_index, W, a):
    e0 = edge_index[0]
    e1 = edge_index[1]
    xp = jnp.pad(x, ((0, NP - N), (0, 0)))
    hpad, q, pa, pb = _node_stage(xp, W, a)
    r2, t = _edge_gather(q, pa, pb, e0, e1)
    ee = _edge_weight(r2.reshape(E // D, D), t.reshape(E // D, D)).reshape(E)
    zero = jnp.zeros((N, DP), jnp.float32)
    acc = _spmm(hpad, e0, e1, ee, zero)
    return _finalize(acc)


# speedup vs baseline: 1.9123x; 1.9123x over previous
import functools

import jax
import jax.numpy as jnp
from jax import lax
from jax.experimental import pallas as pl
from jax.experimental.pallas import tpu as pltpu
from jax.experimental.pallas import tpu_sc as plsc

N = 10000
E = 320000
D = 128
DP = 144
NC = 2
NS = 16
L = 16
NW = NC * NS
EPW = E // NW
K = 80
RPW = N // NS

_MAXNORM = 1.0 - 4e-3

_mesh = plsc.VectorSubcoreMesh(core_axis_name="c", subcore_axis_name="s")


def _artanh(v):
    c = jnp.clip(v, -1.0 + 1e-7, 1.0 - 1e-7)
    return 0.5 * jnp.log((1.0 + c) / (1.0 - c))


def _tanh15(v):
    return jnp.tanh(jnp.clip(v, -15.0, 15.0))


def _rownorm(v):
    return jnp.clip(
        jnp.sqrt(jnp.sum(v * v, axis=-1, keepdims=True)), 1e-15, None)


NP = 10240
BN = 1024


def _node_stage_body(x_ref, w_ref, a_ref, hpad_ref, q_ref, pa_ref, pb_ref):
    x = x_ref[...]
    w = w_ref[...]
    av = a_ref[...]
    xn = _rownorm(x)
    mx = jnp.dot(x, w, preferred_element_type=jnp.float32)
    mxn = _rownorm(mx)
    h1 = _tanh15(mxn / xn * _artanh(xn)) * mx / mxn
    n1 = _rownorm(h1)
    h2 = jnp.where(n1 > _MAXNORM, h1 / n1 * _MAXNORM, h1)
    pn = _rownorm(h2)
    h = h2 / pn * _artanh(pn)
    q_ref[...] = jnp.sum(h * h, axis=-1)
    pa_ref[...] = jnp.sum(h * av[:, :D], axis=-1)
    pb_ref[...] = jnp.sum(h * av[:, D:], axis=-1)
    ones = jnp.ones((h.shape[0], 1), jnp.float32)
    zeros = jnp.zeros((h.shape[0], DP - D - 1), jnp.float32)
    hpad_ref[...] = jnp.concatenate([h, ones, zeros], axis=1)


_node_stage = pl.pallas_call(
    _node_stage_body,
    grid=(NP // BN,),
    in_specs=[
        pl.BlockSpec((BN, D), lambda i: (i, 0)),
        pl.BlockSpec((D, D), lambda i: (0, 0)),
        pl.BlockSpec((1, 2 * D), lambda i: (0, 0)),
    ],
    out_specs=[
        pl.BlockSpec((BN, DP), lambda i: (i, 0)),
        pl.BlockSpec((BN,), lambda i: (i,)),
        pl.BlockSpec((BN,), lambda i: (i,)),
        pl.BlockSpec((BN,), lambda i: (i,)),
    ],
    out_shape=[
        jax.ShapeDtypeStruct((NP, DP), jnp.float32),
        jax.ShapeDtypeStruct((NP,), jnp.float32),
        jax.ShapeDtypeStruct((NP,), jnp.float32),
        jax.ShapeDtypeStruct((NP,), jnp.float32),
    ],
)


@functools.partial(
    pl.kernel,
    out_type=[
        jax.ShapeDtypeStruct((E,), jnp.float32),
        jax.ShapeDtypeStruct((E,), jnp.float32),
    ],
    mesh=_mesh,
    scratch_types=[
        pltpu.VMEM((NP,), jnp.float32),
        pltpu.VMEM((NP,), jnp.float32),
        pltpu.VMEM((NP,), jnp.float32),
        pltpu.VMEM((EPW,), jnp.int32),
        pltpu.VMEM((EPW,), jnp.int32),
        pltpu.VMEM((EPW,), jnp.float32),
        pltpu.VMEM((EPW,), jnp.float32),
        pltpu.SemaphoreType.DMA,
    ],
    compiler_params=pltpu.CompilerParams(needs_layout_passes=False),
)
def _edge_gather(q_hbm, pa_hbm, pb_hbm, e0_hbm, e1_hbm, r2_hbm, t_hbm,
                 qv, pav, pbv, i0v, i1v, r2v, tv, sem):
    wid = lax.axis_index("s") * NC + lax.axis_index("c")
    base = wid * EPW
    pltpu.async_copy(q_hbm, qv, sem)
    pltpu.async_copy(pa_hbm, pav, sem)
    pltpu.async_copy(pb_hbm, pbv, sem)
    pltpu.async_copy(e0_hbm.at[pl.ds(base, EPW)], i0v, sem)
    pltpu.async_copy(e1_hbm.at[pl.ds(base, EPW)], i1v, sem)
    pltpu.make_async_copy(q_hbm, qv, sem).wait()
    pltpu.make_async_copy(pa_hbm, pav, sem).wait()
    pltpu.make_async_copy(pb_hbm, pbv, sem).wait()
    pltpu.make_async_copy(e0_hbm.at[pl.ds(base, EPW)], i0v, sem).wait()
    pltpu.make_async_copy(e1_hbm.at[pl.ds(base, EPW)], i1v, sem).wait()

    def body(i5, carry):
        for u in range(5):
            i = i5 * 5 + u
            sl = pl.ds(i * L, L)
            i0 = i0v[sl]
            i1 = i1v[sl]
            q0 = plsc.load_gather(qv, [i0])
            q1 = plsc.load_gather(qv, [i1])
            p0 = plsc.load_gather(pav, [i0])
            p1 = plsc.load_gather(pbv, [i1])
            r2v[sl] = q0 + q1
            tv[sl] = p0 + p1
        return carry

    lax.fori_loop(0, EPW // L // 5, body, 0)
    pltpu.sync_copy(r2v, r2_hbm.at[pl.ds(base, EPW)])
    pltpu.sync_copy(tv, t_hbm.at[pl.ds(base, EPW)])


def _edge_weight_body(r2_ref, t_ref, ee_ref):
    r2 = r2_ref[...]
    t = t_ref[...]
    un = jnp.sqrt(r2)
    r = jnp.maximum(un, 1e-15)
    em_scale = _tanh15(r) / r
    nu = jnp.maximum(em_scale * un, 1e-15)
    p_scale = jnp.where(nu > _MAXNORM, _MAXNORM / nu, 1.0)
    alpha = em_scale * p_scale
    rho = jnp.maximum(alpha * un, 1e-15)
    mx = alpha * t
    mxn = jnp.maximum(jnp.abs(mx), 1e-15)
    s = _tanh15(mxn / rho * _artanh(rho)) * mx / mxn
    bign = jnp.sqrt(jnp.sum(s * s))
    proj_scale = jnp.where(bign > _MAXNORM, _MAXNORM / bign, 1.0)
    pn = jnp.maximum(bign * proj_scale, 1e-15)
    beta = _artanh(pn) / pn * proj_scale
    sf = s * beta
    gelu = 0.5 * sf * (1.0 + lax.erf(sf / jnp.sqrt(2.0).astype(jnp.float32)))
    ee_ref[...] = jnp.exp(-gelu)


_edge_weight = pl.pallas_call(
    _edge_weight_body,
    out_shape=jax.ShapeDtypeStruct((E // D, D), jnp.float32),
)


NCHK = EPW // K


@functools.partial(
    pl.kernel,
    out_type=jax.ShapeDtypeStruct((NC, N, DP), jnp.float32),
    mesh=_mesh,
    scratch_types=(
        [pltpu.VMEM((K,), jnp.int32)] * 3
        + [pltpu.VMEM((K,), jnp.int32)] * 3
        + [pltpu.VMEM((K,), jnp.float32)] * 3
        + [pltpu.VMEM((K,), jnp.int32)] * 3
        + [pltpu.VMEM((K, DP), jnp.float32)] * 3
        + [pltpu.VMEM_SHARED((N, DP), jnp.float32)]
        + [pltpu.SemaphoreType.DMA] * 9
    ),
    compiler_params=pltpu.CompilerParams(
        needs_layout_passes=False, use_tc_tiling_on_sc=False),
)
def _spmm(hpad_hbm, e0_hbm, e1_hbm, ee_hbm, zero_hbm, out_hbm,
          i0_0, i0_1, i0_2, i1_0, i1_1, i1_2, ee_0, ee_1, ee_2,
          si_0, si_1, si_2, rw_0, rw_1, rw_2, acc,
          g_0, g_1, g_2, c_0, c_1, c_2, s_0, s_1, s_2):
    cid = lax.axis_index("c")
    sid = lax.axis_index("s")
    base = (sid * NC + cid) * EPW
    pltpu.sync_copy(zero_hbm.at[pl.ds(sid * RPW, RPW)],
                    acc.at[pl.ds(sid * RPW, RPW)])
    plsc.subcore_barrier()

    sets = [
        dict(i0=i0_0, i1=i1_0, ee=ee_0, si=si_0, rw=rw_0, g=g_0, c=c_0,
             s=s_0),
        dict(i0=i0_1, i1=i1_1, ee=ee_1, si=si_1, rw=rw_1, g=g_1, c=c_1,
             s=s_1),
        dict(i0=i0_2, i1=i1_2, ee=ee_2, si=si_2, rw=rw_2, g=g_2, c=c_2,
             s=s_2),
    ]

    def issue_idx(k, st):
        off = base + k * K
        pltpu.async_copy(e0_hbm.at[pl.ds(off, K)], st["i0"], st["s"])
        pltpu.async_copy(e1_hbm.at[pl.ds(off, K)], st["i1"], st["s"])
        pltpu.async_copy(ee_hbm.at[pl.ds(off, K)], st["ee"], st["s"])

    def wait_idx(st):
        pltpu.make_async_copy(e0_hbm.at[pl.ds(0, K)], st["i0"], st["s"]).wait()
        pltpu.make_async_copy(e1_hbm.at[pl.ds(0, K)], st["i1"], st["s"]).wait()
        pltpu.make_async_copy(ee_hbm.at[pl.ds(0, K)], st["ee"], st["s"]).wait()

    def wait_scatter(st):
        pltpu.make_async_copy(st["rw"], acc.at[st["si"]], st["c"]).wait()

    def slot(k, cur, nxt, prv):
        @pl.when(k + 1 < NCHK)
        def _():
            wait_idx(nxt)

            @pl.when(k >= 2)
            def _():
                wait_scatter(nxt)

            pltpu.async_copy(hpad_hbm.at[nxt["i1"]], nxt["rw"], nxt["g"])

        buf = cur["rw"]
        ee = cur["ee"]
        pltpu.make_async_copy(hpad_hbm.at[cur["i1"]], buf, cur["g"]).wait()

        def _scale(r4, c2):
            for u in range(4):
                r = r4 * 4 + u
                eb = plsc.load_gather(ee, [jnp.full((L,), 0, jnp.int32) + r])
                for j in range(DP // L):
                    sl = pl.ds(j * L, L)
                    buf[r, sl] = buf[r, sl] * eb
            return c2

        lax.fori_loop(0, K // 4, _scale, 0)

        for v in range(K // L):
            sl = pl.ds(v * L, L)
            cur["si"][sl] = cur["i0"][sl]

        @pl.when(k + 2 < NCHK)
        def _():
            issue_idx(k + 2, prv)

        pltpu.async_copy(buf, acc.at[cur["si"]], cur["c"], add=True)

    issue_idx(0, sets[0])
    issue_idx(1, sets[1])
    wait_idx(sets[0])
    pltpu.async_copy(hpad_hbm.at[sets[0]["i1"]], sets[0]["rw"], sets[0]["g"])

    def triple(k3, carry):
        k = 3 * k3
        slot(k, sets[0], sets[1], sets[2])

        @pl.when(k + 1 < NCHK)
        def _():
            slot(k + 1, sets[1], sets[2], sets[0])

        @pl.when(k + 2 < NCHK)
        def _():
            slot(k + 2, sets[2], sets[0], sets[1])

        return carry

    lax.fori_loop(0, (NCHK + 2) // 3, triple, 0)
    wait_scatter(sets[(NCHK - 3) % 3])
    wait_scatter(sets[(NCHK - 2) % 3])
    wait_scatter(sets[(NCHK - 1) % 3])
    plsc.subcore_barrier()
    pltpu.sync_copy(acc.at[pl.ds(sid * RPW, RPW)],
                    out_hbm.at[cid, pl.ds(sid * RPW, RPW)])


def _finalize_body(acc_ref, out_ref):
    A = acc_ref[0] + acc_ref[1]
    hp = A[:, :D]
    col = lax.broadcasted_iota(jnp.int32, A.shape, 1)
    rs = jnp.sum(jnp.where(col == D, A, 0.0), axis=-1, keepdims=True)
    v = jax.nn.relu(hp / rs)
    un = _rownorm(v)
    em = _tanh15(un) * v / un
    n1 = _rownorm(em)
    out_ref[...] = jnp.where(n1 > _MAXNORM, em / n1 * _MAXNORM, em)


_finalize = pl.pallas_call(
    _finalize_body,
    out_shape=jax.ShapeDtypeStruct((N, D), jnp.float32),
)


def kernel(x, edge_index, W, a):
    e0 = edge_index[0]
    e1 = edge_index[1]
    xp = jnp.pad(x, ((0, NP - N), (0, 0)))
    hpad, q, pa, pb = _node_stage(xp, W, a)
    r2, t = _edge_gather(q, pa, pb, e0, e1)
    ee = _edge_weight(r2.reshape(E // D, D), t.reshape(E // D, D)).reshape(E)
    zero = jnp.zeros((N, DP), jnp.float32)
    acc = _spmm(hpad, e0, e1, ee, zero)
    return _finalize(acc)
